# Initial kernel scaffold; baseline (speedup 1.0000x reference)
import jax
import jax.numpy as jnp
from jax.experimental import pallas as pl


def kernel(x, edge_index, edge_attr, x_clique, atom2clique_row, atom2clique_col, tree_edge_index, batch, tree_batch, bond_W, bond_b, eps, mlp_W1, mlp_b1, bn1_g, bn1_b, mlp_W2, mlp_b2, abn_g, abn_b, a2c_W, a2c_b, ceps, cconv_W, cconv_b, cbn_g, cbn_b, c2a_W, c2a_b, atom_lin_W, atom_lin_b, clique_lin_W, clique_lin_b, lin_W, lin_b):
    raise NotImplementedError("write your pallas kernel here")


# speedup vs baseline: 2.2939x; 2.2939x over previous
import functools

import jax
import jax.numpy as jnp
from jax import lax
from jax.experimental import pallas as pl
from jax.experimental.pallas import tpu as pltpu
from jax.experimental.pallas import tpu_sc as plsc

NC, NS, LN = 2, 16, 16
NW = NC * NS
H = 128
N, E, C, G, L = 10000, 320000, 3000, 256, 3

F32 = jnp.float32


def _mesh():
    return plsc.VectorSubcoreMesh(core_axis_name="c", subcore_axis_name="s",
                                  num_cores=NC, num_subcores=NS)


def _gather_scatter(table, src, dst, S_pad, chunk, ea=None):
    K_tot = src.shape[0]
    PW = K_tot // NW
    nch = PW // chunk
    assert PW % chunk == 0 and chunk % 8 == 0 and chunk <= 128
    S16 = S_pad // NS
    has_ea = ea is not None

    scratch = [
        pltpu.VMEM((chunk,), jnp.int32),
        pltpu.VMEM((chunk,), jnp.int32),
        pltpu.VMEM((chunk, H), F32),
    ]
    if has_ea:
        scratch.append(pltpu.VMEM((chunk, H), F32))
    scratch += [
        pltpu.VMEM_SHARED((S_pad, H), F32),
        pltpu.SemaphoreType.DMA,
    ]

    def body(*refs):
        if has_ea:
            (tab, src_h, dst_h, ea_h, zeros_h, out_h,
             src_v, dst_v, rows_v, ea_v, acc, sem) = refs
        else:
            (tab, src_h, dst_h, zeros_h, out_h,
             src_v, dst_v, rows_v, acc, sem) = refs
        cid = lax.axis_index("c")
        sid = lax.axis_index("s")
        wid = sid * NC + cid
        base = wid * PW

        pltpu.sync_copy(zeros_h, acc.at[pl.ds(sid * S16, S16)])
        plsc.subcore_barrier()

        def step(g, carry):
            off = base + g * chunk
            pltpu.sync_copy(src_h.at[pl.ds(off, chunk)], src_v)
            pltpu.sync_copy(dst_h.at[pl.ds(off, chunk)], dst_v)
            pltpu.async_copy(tab.at[src_v], rows_v, sem).wait()
            if has_ea:
                pltpu.sync_copy(ea_h.at[pl.ds(off, chunk)], ea_v)

                def rbody(r, c2):
                    for j in range(H // LN):
                        sl = pl.ds(j * LN, LN)
                        rows_v[r, sl] = jnp.maximum(rows_v[r, sl] + ea_v[r, sl],
                                                    0.0)
                    return c2

                lax.fori_loop(0, chunk, rbody, 0)
            pltpu.sync_copy(rows_v, acc.at[dst_v], add=True)
            return carry

        lax.fori_loop(0, nch, step, 0)
        plsc.subcore_barrier()
        pltpu.sync_copy(acc.at[pl.ds(sid * S16, S16)],
                        out_h.at[pl.ds(cid * S_pad + sid * S16, S16)])

    zeros = jnp.zeros((S16, H), F32)
    k = pl.kernel(body, out_type=jax.ShapeDtypeStruct((NC * S_pad, H), F32),
                  mesh=_mesh(), scratch_types=scratch)
    if has_ea:
        out = k(table, src, dst, ea, zeros)
    else:
        out = k(table, src, dst, zeros)
    return out.reshape(NC, S_pad, H)


def _counts(col_s, row_s, bat_p, tb_p):
    specs = [
        (col_s, 3072, 320, 80),
        (row_s, 10112, 320, 80),
        (bat_p, 384, 320, 80),
        (tb_p, 384, 96, 48),
    ]
    CW = 128
    scratch = [pltpu.VMEM((80, CW), F32),
               pltpu.VMEM((80,), jnp.int32),
               pltpu.VMEM((48,), jnp.int32)]
    scratch += [pltpu.VMEM_SHARED((sp, CW), F32) for _, sp, _, _ in specs]

    def body(c_h, r_h, b_h, t_h, ones_h, zeros_h, o1, o2, o3, o4,
             ones_v, d80, d48, a1, a2, a3, a4):
        cid = lax.axis_index("c")
        sid = lax.axis_index("s")
        wid = sid * NC + cid
        dsts = [c_h, r_h, b_h, t_h]
        outs = [o1, o2, o3, o4]
        accs = [a1, a2, a3, a4]
        pltpu.sync_copy(ones_h, ones_v)
        for (_, sp, _, _), acc in zip(specs, accs):
            s16 = sp // NS
            pltpu.sync_copy(zeros_h.at[pl.ds(0, s16)],
                            acc.at[pl.ds(sid * s16, s16)])
        plsc.subcore_barrier()
        for (_, sp, pw, ch), d_h, acc in zip(specs, dsts, accs):
            dv = d80 if ch == 80 else d48
            base = wid * pw

            def step(g, carry, d_h=d_h, acc=acc, dv=dv, ch=ch, base=base):
                pltpu.sync_copy(d_h.at[pl.ds(base + g * ch, ch)], dv)
                pltpu.sync_copy(ones_v.at[pl.ds(0, ch)], acc.at[dv], add=True)
                return carry

            lax.fori_loop(0, pw // ch, step, 0)
        plsc.subcore_barrier()
        for (_, sp, _, _), out_h, acc in zip(specs, outs, accs):
            s16 = sp // NS
            pltpu.sync_copy(acc.at[pl.ds(sid * s16, s16)],
                            out_h.at[pl.ds(cid * sp + sid * s16, s16)])

    ones = jnp.ones((80, CW), F32)
    zeros = jnp.zeros((10112 // NS, CW), F32)
    outs = tuple(jax.ShapeDtypeStruct((NC * sp, CW), F32)
                 for _, sp, _, _ in specs)
    k = pl.kernel(body, out_type=outs, mesh=_mesh(), scratch_types=scratch)
    o = k(col_s, row_s, bat_p, tb_p, ones, zeros)
    res = []
    for (_, sp, _, _), oi in zip(specs, o):
        p = oi.reshape(NC, sp, CW)
        res.append(p[0, :, 0:1] + p[1, :, 0:1])
    return res


def _tc(f, out_shape, *args):
    return pl.pallas_call(f, out_shape=out_shape)(*args)


def _ea_body(attr_ref, w_ref, b_ref, out_ref):
    out_ref[...] = jnp.dot(attr_ref[...], w_ref[...], preferred_element_type=F32,
                           precision=jax.lax.Precision.HIGHEST) + b_ref[0][None, :]


def _compute_ea(edge_attr, bond_Wi, bond_bi):
    BE = 8000
    return pl.pallas_call(
        _ea_body,
        grid=(E // BE,),
        in_specs=[
            pl.BlockSpec((BE, 16), lambda j: (j, 0)),
            pl.BlockSpec((16, H), lambda j: (0, 0)),
            pl.BlockSpec((1, H), lambda j: (0, 0)),
        ],
        out_specs=pl.BlockSpec((BE, H), lambda j: (j, 0)),
        out_shape=jax.ShapeDtypeStruct((E, H), F32),
    )(edge_attr, bond_Wi, bond_bi)


def _bn(z, g, b):
    m = jnp.mean(z, 0, keepdims=True)
    v = jnp.mean((z - m) ** 2, 0, keepdims=True)
    return (z - m) * jax.lax.rsqrt(v + 1e-5) * g[None, :] + b[None, :]


def _mlp_body(x_ref, agg_ref, eps_ref, w1_ref, b1_ref, g1_ref, bb1_ref,
              w2_ref, b2_ref, ga_ref, ba_ref, out_ref):
    h = x_ref[...] * (1.0 + eps_ref[0, 0]) + agg_ref[0, :N] + agg_ref[1, :N]
    z = jnp.dot(h, w1_ref[...], preferred_element_type=F32,
                precision=jax.lax.Precision.HIGHEST) + b1_ref[0][None, :]
    z = jnp.maximum(_bn(z, g1_ref[0], bb1_ref[0]), 0.0)
    z = jnp.dot(z, w2_ref[...], preferred_element_type=F32,
                precision=jax.lax.Precision.HIGHEST) + b2_ref[0][None, :]
    out_ref[...] = jnp.maximum(_bn(z, ga_ref[0], ba_ref[0]), 0.0)


def _a2c_body(xc_ref, tp_ref, cp_ref, w_ref, b_ref, out_ref):
    t = (tp_ref[0, :C] + tp_ref[1, :C]) / jnp.maximum(cp_ref[:C], 1.0)
    out_ref[...] = xc_ref[...] + jnp.maximum(
        jnp.dot(t, w_ref[...], preferred_element_type=F32,
                precision=jax.lax.Precision.HIGHEST) + b_ref[0][None, :],
        0.0)


def _cconv_body(xc_ref, ap_ref, ceps_ref, w_ref, b_ref, g_ref, bb_ref, out_ref):
    agg = ap_ref[0, :C] + ap_ref[1, :C]
    z = xc_ref[...] * (1.0 + ceps_ref[0, 0]) + agg
    z = jnp.dot(z, w_ref[...], preferred_element_type=F32,
                precision=jax.lax.Precision.HIGHEST) + b_ref[0][None, :]
    out_ref[...] = jnp.maximum(_bn(z, g_ref[0], bb_ref[0]), 0.0)


def _c2a_body(x_ref, mp_ref, cp_ref, w_ref, b_ref, out_ref):
    m = (mp_ref[0, :N] + mp_ref[1, :N]) / jnp.maximum(cp_ref[:N], 1.0)
    out_ref[...] = x_ref[...] + jnp.maximum(
        jnp.dot(m, w_ref[...], preferred_element_type=F32,
                precision=jax.lax.Precision.HIGHEST) + b_ref[0][None, :],
        0.0)


def _final_body(gp_ref, gc_ref, tgp_ref, tgc_ref, aw_ref, ab_ref,
                cw_ref, cb_ref, lw_ref, lb_ref, out_ref):
    xg = (gp_ref[0, :G] + gp_ref[1, :G]) / jnp.maximum(gc_ref[:G], 1.0)
    xg = jnp.dot(xg, aw_ref[...], preferred_element_type=F32,
                precision=jax.lax.Precision.HIGHEST) + ab_ref[0][None, :]
    xcg = (tgp_ref[0, :G] + tgp_ref[1, :G]) / jnp.maximum(tgc_ref[:G], 1.0)
    xcg = (jnp.dot(xcg, cw_ref[...], preferred_element_type=F32,
                precision=jax.lax.Precision.HIGHEST)
           + cb_ref[0][None, :])
    y = jnp.maximum(xg + xcg, 0.0)
    out_ref[...] = jnp.dot(y, lw_ref[...], preferred_element_type=F32,
                precision=jax.lax.Precision.HIGHEST) \
        + lb_ref[0][None, :]


def _pad(a, n, v):
    return jnp.concatenate([a, jnp.full((n - a.shape[0],), v, a.dtype)])


def kernel(x, edge_index, edge_attr, x_clique, atom2clique_row,
           atom2clique_col, tree_edge_index, batch, tree_batch,
           bond_W, bond_b, eps, mlp_W1, mlp_b1, bn1_g, bn1_b, mlp_W2, mlp_b2,
           abn_g, abn_b, a2c_W, a2c_b, ceps, cconv_W, cconv_b, cbn_g, cbn_b,
           c2a_W, c2a_b, atom_lin_W, atom_lin_b, clique_lin_W, clique_lin_b,
           lin_W, lin_b):
    src, dst = edge_index[0], edge_index[1]
    tsrc, tdst = tree_edge_index[0], tree_edge_index[1]
    row, col = atom2clique_row, atom2clique_col

    row_g, row_s = _pad(row, 10240, 0), _pad(row, 10240, N)
    col_g, col_s = _pad(col, 10240, 0), _pad(col, 10240, C)
    tsrc_p, tdst_p = _pad(tsrc, 6144, 0), _pad(tdst, 6144, C)
    bat_p = _pad(batch, 10240, G)
    tb_p = _pad(tree_batch, 3072, G)
    ar_n = _pad(jnp.arange(N, dtype=jnp.int32), 10240, 0)
    ar_c = _pad(jnp.arange(C, dtype=jnp.int32), 3072, 0)

    cnt_col, cnt_row, cnt_bat, cnt_tb = _counts(col_s, row_s, bat_p, tb_p)

    for i in range(L):
        ea = _compute_ea(edge_attr, bond_W[i], bond_b[i:i + 1])
        agg = _gather_scatter(x, src, dst, 10112, 80, ea=ea)
        x = _tc(_mlp_body, jax.ShapeDtypeStruct((N, H), F32),
                x, agg, eps[i].reshape(1, 1), mlp_W1[i], mlp_b1[i:i + 1],
                bn1_g[i:i + 1], bn1_b[i:i + 1], mlp_W2[i], mlp_b2[i:i + 1],
                abn_g[i:i + 1], abn_b[i:i + 1])

        tp = _gather_scatter(x, row_g, col_s, 3072, 80)
        x_clique = _tc(_a2c_body, jax.ShapeDtypeStruct((C, H), F32),
                       x_clique, tp, cnt_col, a2c_W[i], a2c_b[i:i + 1])

        ap = _gather_scatter(x_clique, tsrc_p, tdst_p, 3072, 64)
        x_clique = _tc(_cconv_body, jax.ShapeDtypeStruct((C, H), F32),
                       x_clique, ap, ceps[i].reshape(1, 1), cconv_W[i],
                       cconv_b[i:i + 1], cbn_g[i:i + 1], cbn_b[i:i + 1])

        mp = _gather_scatter(x_clique, col_g, row_s, 10112, 80)
        x = _tc(_c2a_body, jax.ShapeDtypeStruct((N, H), F32),
                x, mp, cnt_row, c2a_W[i], c2a_b[i:i + 1])

    gp = _gather_scatter(x, ar_n, bat_p, 384, 80)
    tgp = _gather_scatter(x_clique, ar_c, tb_p, 384, 48)
    out = _tc(_final_body, jax.ShapeDtypeStruct((G, 1), F32),
              gp, cnt_bat, tgp, cnt_tb, atom_lin_W, atom_lin_b.reshape(1, H),
              clique_lin_W, clique_lin_b.reshape(1, H), lin_W,
              lin_b.reshape(1, 1))
    return out


# speedup vs baseline: 3.1733x; 1.3834x over previous
import functools

import jax
import jax.numpy as jnp
from jax import lax
from jax.experimental import pallas as pl
from jax.experimental.pallas import tpu as pltpu
from jax.experimental.pallas import tpu_sc as plsc

NC, NS, LN = 2, 16, 16
NW = NC * NS
H = 128
N, E, C, G, L = 10000, 320000, 3000, 256, 3

F32 = jnp.float32


def _mesh():
    return plsc.VectorSubcoreMesh(core_axis_name="c", subcore_axis_name="s",
                                  num_cores=NC, num_subcores=NS)


def _gather_scatter(table, src, dst, S_pad, chunk, ea=None):
    K_tot = src.shape[0]
    PW = K_tot // NW
    nch = PW // chunk
    assert PW % chunk == 0 and chunk % 8 == 0 and chunk <= 128
    S16 = S_pad // NS
    has_ea = ea is not None

    scratch = [
        pltpu.VMEM((chunk,), jnp.int32),
        pltpu.VMEM((chunk,), jnp.int32),
        pltpu.VMEM((chunk, H), F32),
    ]
    if has_ea:
        scratch.append(pltpu.VMEM((chunk, H), F32))
    scratch += [
        pltpu.VMEM_SHARED((S_pad, H), F32),
        pltpu.SemaphoreType.DMA,
    ]

    def body(*refs):
        if has_ea:
            (tab, src_h, dst_h, ea_h, zeros_h, out_h,
             src_v, dst_v, rows_v, ea_v, acc, sem) = refs
        else:
            (tab, src_h, dst_h, zeros_h, out_h,
             src_v, dst_v, rows_v, acc, sem) = refs
        cid = lax.axis_index("c")
        sid = lax.axis_index("s")
        wid = sid * NC + cid
        base = wid * PW

        pltpu.sync_copy(zeros_h, acc.at[pl.ds(sid * S16, S16)])
        plsc.subcore_barrier()

        def step(g, carry):
            off = base + g * chunk
            pltpu.sync_copy(src_h.at[pl.ds(off, chunk)], src_v)
            pltpu.sync_copy(dst_h.at[pl.ds(off, chunk)], dst_v)
            pltpu.async_copy(tab.at[src_v], rows_v, sem).wait()
            if has_ea:
                pltpu.sync_copy(ea_h.at[pl.ds(off, chunk)], ea_v)

                def rbody(r, c2):
                    for j in range(H // LN):
                        sl = pl.ds(j * LN, LN)
                        rows_v[r, sl] = jnp.maximum(rows_v[r, sl] + ea_v[r, sl],
                                                    0.0)
                    return c2

                lax.fori_loop(0, chunk, rbody, 0)
            pltpu.sync_copy(rows_v, acc.at[dst_v], add=True)
            return carry

        lax.fori_loop(0, nch, step, 0)
        plsc.subcore_barrier()
        pltpu.sync_copy(acc.at[pl.ds(sid * S16, S16)],
                        out_h.at[pl.ds(cid * S_pad + sid * S16, S16)])

    zeros = jnp.zeros((S16, H), F32)
    k = pl.kernel(body, out_type=jax.ShapeDtypeStruct((NC * S_pad, H), F32),
                  mesh=_mesh(), scratch_types=scratch)
    if has_ea:
        out = k(table, src, dst, ea, zeros)
    else:
        out = k(table, src, dst, zeros)
    return out.reshape(NC, S_pad, H)


def _edge_agg(x, src, dst2, ea, S_pad=10112, K=80):
    PW = E // NW
    nch = PW // K
    S16 = S_pad // NS

    scratch = [
        pltpu.VMEM((PW,), jnp.int32),
        pltpu.VMEM((nch, K), jnp.int32),
        pltpu.VMEM((2 * K, H), F32),
        pltpu.VMEM_SHARED((S_pad, H), F32),
        pltpu.SemaphoreType.DMA,
    ]

    def body(tab, src_h, dst_h, ea_h, zeros_h, out_h,
             src_v, dst_v, buf_v, acc, gsem):
        cid = lax.axis_index("c")
        sid = lax.axis_index("s")
        wid = sid * NC + cid
        base = wid * PW

        pltpu.sync_copy(zeros_h, acc.at[pl.ds(sid * S16, S16)])
        pltpu.sync_copy(src_h.at[pl.ds(base, PW)], src_v)
        pltpu.sync_copy(dst_h.at[wid], dst_v)
        plsc.subcore_barrier()

        def fire(g, slot):
            off = g * K
            dslot = buf_v.at[pl.ds(slot * K, K)]
            pltpu.sync_copy(ea_h.at[pl.ds(base + off, K)], dslot)
            pltpu.async_copy(tab.at[src_v.at[pl.ds(off, K)]], dslot, gsem,
                             add=True)

        fire(0, 0)

        def step(g, carry):
            slot = lax.rem(g, 2)
            dslot = buf_v.at[pl.ds(slot * K, K)]
            pltpu.make_async_copy(tab.at[src_v.at[pl.ds(0, K)]], dslot,
                                  gsem).wait()

            @pl.when(g + 1 < nch)
            def _():
                fire(g + 1, 1 - slot)

            rbase = slot * K

            def rbody(r, c2):
                for j in range(H // LN):
                    sl = pl.ds(j * LN, LN)
                    buf_v[rbase + r, sl] = jnp.maximum(buf_v[rbase + r, sl],
                                                       0.0)
                return c2

            plsc.parallel_loop(0, K, 1, unroll=4, carry=None)(
                lambda r: rbody(r, None))
            pltpu.sync_copy(dslot, acc.at[dst_v.at[g]], add=True)
            return carry

        lax.fori_loop(0, nch, step, 0)
        plsc.subcore_barrier()
        pltpu.sync_copy(acc.at[pl.ds(sid * S16, S16)],
                        out_h.at[pl.ds(cid * S_pad + sid * S16, S16)])

    zeros = jnp.zeros((S16, H), F32)
    k = pl.kernel(body, out_type=jax.ShapeDtypeStruct((NC * S_pad, H), F32),
                  mesh=_mesh(), scratch_types=scratch)
    return k(x, src, dst2, ea, zeros).reshape(NC, S_pad, H)


def _counts(col_s, row_s, bat_p, tb_p):
    specs = [
        (col_s, 3072, 320, 80),
        (row_s, 10112, 320, 80),
        (bat_p, 384, 320, 80),
        (tb_p, 384, 96, 48),
    ]
    CW = 128
    scratch = [pltpu.VMEM((80, CW), F32),
               pltpu.VMEM((80,), jnp.int32),
               pltpu.VMEM((48,), jnp.int32)]
    scratch += [pltpu.VMEM_SHARED((sp, CW), F32) for _, sp, _, _ in specs]

    def body(c_h, r_h, b_h, t_h, ones_h, zeros_h, o1, o2, o3, o4,
             ones_v, d80, d48, a1, a2, a3, a4):
        cid = lax.axis_index("c")
        sid = lax.axis_index("s")
        wid = sid * NC + cid
        dsts = [c_h, r_h, b_h, t_h]
        outs = [o1, o2, o3, o4]
        accs = [a1, a2, a3, a4]
        pltpu.sync_copy(ones_h, ones_v)
        for (_, sp, _, _), acc in zip(specs, accs):
            s16 = sp // NS
            pltpu.sync_copy(zeros_h.at[pl.ds(0, s16)],
                            acc.at[pl.ds(sid * s16, s16)])
        plsc.subcore_barrier()
        for (_, sp, pw, ch), d_h, acc in zip(specs, dsts, accs):
            dv = d80 if ch == 80 else d48
            base = wid * pw

            def step(g, carry, d_h=d_h, acc=acc, dv=dv, ch=ch, base=base):
                pltpu.sync_copy(d_h.at[pl.ds(base + g * ch, ch)], dv)
                pltpu.sync_copy(ones_v.at[pl.ds(0, ch)], acc.at[dv], add=True)
                return carry

            lax.fori_loop(0, pw // ch, step, 0)
        plsc.subcore_barrier()
        for (_, sp, _, _), out_h, acc in zip(specs, outs, accs):
            s16 = sp // NS
            pltpu.sync_copy(acc.at[pl.ds(sid * s16, s16)],
                            out_h.at[pl.ds(cid * sp + sid * s16, s16)])

    ones = jnp.ones((80, CW), F32)
    zeros = jnp.zeros((10112 // NS, CW), F32)
    outs = tuple(jax.ShapeDtypeStruct((NC * sp, CW), F32)
                 for _, sp, _, _ in specs)
    k = pl.kernel(body, out_type=outs, mesh=_mesh(), scratch_types=scratch)
    o = k(col_s, row_s, bat_p, tb_p, ones, zeros)
    res = []
    for (_, sp, _, _), oi in zip(specs, o):
        p = oi.reshape(NC, sp, CW)
        res.append(p[0, :, 0:1] + p[1, :, 0:1])
    return res


def _tc(f, out_shape, *args):
    return pl.pallas_call(f, out_shape=out_shape)(*args)


def _ea_body(attr_ref, w_ref, b_ref, out_ref):
    out_ref[...] = jnp.dot(attr_ref[...], w_ref[...], preferred_element_type=F32,
                           precision=jax.lax.Precision.HIGHEST) + b_ref[0][None, :]


def _compute_ea(edge_attr, bond_Wi, bond_bi):
    BE = 8000
    return pl.pallas_call(
        _ea_body,
        grid=(E // BE,),
        in_specs=[
            pl.BlockSpec((BE, 16), lambda j: (j, 0)),
            pl.BlockSpec((16, H), lambda j: (0, 0)),
            pl.BlockSpec((1, H), lambda j: (0, 0)),
        ],
        out_specs=pl.BlockSpec((BE, H), lambda j: (j, 0)),
        out_shape=jax.ShapeDtypeStruct((E, H), F32),
    )(edge_attr, bond_Wi, bond_bi)


def _bn(z, g, b):
    m = jnp.mean(z, 0, keepdims=True)
    v = jnp.mean((z - m) ** 2, 0, keepdims=True)
    return (z - m) / jnp.sqrt(v + 1e-5) * g[None, :] + b[None, :]


def _mlp_body(x_ref, agg_ref, eps_ref, w1_ref, b1_ref, g1_ref, bb1_ref,
              w2_ref, b2_ref, ga_ref, ba_ref, out_ref):
    h = x_ref[...] * (1.0 + eps_ref[0, 0]) + agg_ref[0, :N] + agg_ref[1, :N]
    z = jnp.dot(h, w1_ref[...], preferred_element_type=F32,
                precision=jax.lax.Precision.HIGHEST) + b1_ref[0][None, :]
    z = jnp.maximum(_bn(z, g1_ref[0], bb1_ref[0]), 0.0)
    z = jnp.dot(z, w2_ref[...], preferred_element_type=F32,
                precision=jax.lax.Precision.HIGHEST) + b2_ref[0][None, :]
    out_ref[...] = jnp.maximum(_bn(z, ga_ref[0], ba_ref[0]), 0.0)


def _a2c_body(xc_ref, tp_ref, cp_ref, w_ref, b_ref, out_ref):
    t = (tp_ref[0, :C] + tp_ref[1, :C]) / jnp.maximum(cp_ref[:C], 1.0)
    out_ref[...] = xc_ref[...] + jnp.maximum(
        jnp.dot(t, w_ref[...], preferred_element_type=F32,
                precision=jax.lax.Precision.HIGHEST) + b_ref[0][None, :],
        0.0)


def _cconv_body(xc_ref, ap_ref, ceps_ref, w_ref, b_ref, g_ref, bb_ref, out_ref):
    agg = ap_ref[0, :C] + ap_ref[1, :C]
    z = xc_ref[...] * (1.0 + ceps_ref[0, 0]) + agg
    z = jnp.dot(z, w_ref[...], preferred_element_type=F32,
                precision=jax.lax.Precision.HIGHEST) + b_ref[0][None, :]
    out_ref[...] = jnp.maximum(_bn(z, g_ref[0], bb_ref[0]), 0.0)


def _c2a_body(x_ref, mp_ref, cp_ref, w_ref, b_ref, out_ref):
    m = (mp_ref[0, :N] + mp_ref[1, :N]) / jnp.maximum(cp_ref[:N], 1.0)
    out_ref[...] = x_ref[...] + jnp.maximum(
        jnp.dot(m, w_ref[...], preferred_element_type=F32,
                precision=jax.lax.Precision.HIGHEST) + b_ref[0][None, :],
        0.0)


def _final_body(gp_ref, gc_ref, tgp_ref, tgc_ref, aw_ref, ab_ref,
                cw_ref, cb_ref, lw_ref, lb_ref, out_ref):
    xg = (gp_ref[0, :G] + gp_ref[1, :G]) / jnp.maximum(gc_ref[:G], 1.0)
    xg = jnp.dot(xg, aw_ref[...], preferred_element_type=F32,
                precision=jax.lax.Precision.HIGHEST) + ab_ref[0][None, :]
    xcg = (tgp_ref[0, :G] + tgp_ref[1, :G]) / jnp.maximum(tgc_ref[:G], 1.0)
    xcg = (jnp.dot(xcg, cw_ref[...], preferred_element_type=F32,
                precision=jax.lax.Precision.HIGHEST)
           + cb_ref[0][None, :])
    y = jnp.maximum(xg + xcg, 0.0)
    out_ref[...] = jnp.dot(y, lw_ref[...], preferred_element_type=F32,
                precision=jax.lax.Precision.HIGHEST) \
        + lb_ref[0][None, :]


def _pad(a, n, v):
    return jnp.concatenate([a, jnp.full((n - a.shape[0],), v, a.dtype)])


def kernel(x, edge_index, edge_attr, x_clique, atom2clique_row,
           atom2clique_col, tree_edge_index, batch, tree_batch,
           bond_W, bond_b, eps, mlp_W1, mlp_b1, bn1_g, bn1_b, mlp_W2, mlp_b2,
           abn_g, abn_b, a2c_W, a2c_b, ceps, cconv_W, cconv_b, cbn_g, cbn_b,
           c2a_W, c2a_b, atom_lin_W, atom_lin_b, clique_lin_W, clique_lin_b,
           lin_W, lin_b):
    src, dst = edge_index[0], edge_index[1]
    dst2 = dst.reshape(NW, E // (NW * 80), 80)
    tsrc, tdst = tree_edge_index[0], tree_edge_index[1]
    row, col = atom2clique_row, atom2clique_col

    row_g, row_s = _pad(row, 10240, 0), _pad(row, 10240, N)
    col_g, col_s = _pad(col, 10240, 0), _pad(col, 10240, C)
    tsrc_p, tdst_p = _pad(tsrc, 6144, 0), _pad(tdst, 6144, C)
    bat_p = _pad(batch, 10240, G)
    tb_p = _pad(tree_batch, 3072, G)
    ar_n = _pad(jnp.arange(N, dtype=jnp.int32), 10240, 0)
    ar_c = _pad(jnp.arange(C, dtype=jnp.int32), 3072, 0)

    cnt_col, cnt_row, cnt_bat, cnt_tb = _counts(col_s, row_s, bat_p, tb_p)

    for i in range(L):
        ea = _compute_ea(edge_attr, bond_W[i], bond_b[i:i + 1])
        agg = _edge_agg(x, src, dst2, ea)
        x = _tc(_mlp_body, jax.ShapeDtypeStruct((N, H), F32),
                x, agg, eps[i].reshape(1, 1), mlp_W1[i], mlp_b1[i:i + 1],
                bn1_g[i:i + 1], bn1_b[i:i + 1], mlp_W2[i], mlp_b2[i:i + 1],
                abn_g[i:i + 1], abn_b[i:i + 1])

        tp = _gather_scatter(x, row_g, col_s, 3072, 80)
        x_clique = _tc(_a2c_body, jax.ShapeDtypeStruct((C, H), F32),
                       x_clique, tp, cnt_col, a2c_W[i], a2c_b[i:i + 1])

        ap = _gather_scatter(x_clique, tsrc_p, tdst_p, 3072, 64)
        x_clique = _tc(_cconv_body, jax.ShapeDtypeStruct((C, H), F32),
                       x_clique, ap, ceps[i].reshape(1, 1), cconv_W[i],
                       cconv_b[i:i + 1], cbn_g[i:i + 1], cbn_b[i:i + 1])

        mp = _gather_scatter(x_clique, col_g, row_s, 10112, 80)
        x = _tc(_c2a_body, jax.ShapeDtypeStruct((N, H), F32),
                x, mp, cnt_row, c2a_W[i], c2a_b[i:i + 1])

    gp = _gather_scatter(x, ar_n, bat_p, 384, 80)
    tgp = _gather_scatter(x_clique, ar_c, tb_p, 384, 48)
    out = _tc(_final_body, jax.ShapeDtypeStruct((G, 1), F32),
              gp, cnt_bat, tgp, cnt_tb, atom_lin_W, atom_lin_b.reshape(1, H),
              clique_lin_W, clique_lin_b.reshape(1, H), lin_W,
              lin_b.reshape(1, 1))
    return out
